# Initial kernel scaffold; baseline (speedup 1.0000x reference)
import jax
import jax.numpy as jnp
from jax.experimental import pallas as pl


def kernel(x, emb, Wr, We, be, Al, Bl, Wfc, bfc):
    raise NotImplementedError("write your pallas kernel here")


# speedup vs baseline: 1.3134x; 1.3134x over previous
import functools

import jax
import jax.numpy as jnp
from jax import lax
from jax.experimental import pallas as pl
from jax.experimental.pallas import tpu as pltpu
from jax.experimental.pallas import tpu_sc as plsc

_VOCAB = 32000
_D = 1024
_E = 8
_RANK = 4
_ALPHA = 1.0
_N = 2048

_NC = 2
_NS = 16
_NW = _NC * _NS
_BPW = _N // _NW

_VB = 640
_NT = 1024
_NEG = -1e30


def _emb_gather(x_flat, emb):
    mesh = plsc.VectorSubcoreMesh(core_axis_name="c", subcore_axis_name="s")

    @functools.partial(
        pl.kernel,
        mesh=mesh,
        out_type=jax.ShapeDtypeStruct((_N, _D), jnp.float32),
        scratch_types=[
            pltpu.VMEM((_BPW,), jnp.int32),
            pltpu.VMEM((_BPW, _D), jnp.float32),
            pltpu.SemaphoreType.DMA,
        ],
    )
    def gather_k(idx_hbm, table_hbm, out_hbm, idx_v, rows_v, sem):
        wid = lax.axis_index("s") * _NC + lax.axis_index("c")
        base = wid * _BPW
        pltpu.sync_copy(idx_hbm.at[pl.ds(base, _BPW)], idx_v)
        pltpu.async_copy(table_hbm.at[idx_v], rows_v, sem).wait()
        pltpu.sync_copy(rows_v, out_hbm.at[pl.ds(base, _BPW)])

    return gather_k(x_flat, emb)


def _moe_body(hf_ref, wr_ref, we_ref, be_ref, al_ref, bl_ref, out_ref,
              i1_s, i2_s, p1_s, p2_s, hfb_s):
    e = pl.program_id(1)

    @pl.when(e == 0)
    def _router():
        hf = hf_ref[...]
        hfb = hf.astype(jnp.bfloat16)
        hfb_s[...] = hfb
        logits = lax.dot_general(
            hfb, wr_ref[...].astype(jnp.bfloat16), (((1,), (0,)), ((), ())),
            preferred_element_type=jnp.float32,
        )
        idx = lax.broadcasted_iota(jnp.int32, (_NT, _E), 1)
        m1 = jnp.max(logits, axis=1, keepdims=True)
        i1 = jnp.min(jnp.where(logits == m1, idx, _E), axis=1, keepdims=True)
        l2 = jnp.where(idx == i1, _NEG, logits)
        m2 = jnp.max(l2, axis=1, keepdims=True)
        i2 = jnp.min(jnp.where(l2 == m2, idx, _E), axis=1, keepdims=True)
        p1 = 1.0 / (1.0 + jnp.exp(m2 - m1))
        i1_s[...] = i1
        i2_s[...] = i2
        p1_s[...] = p1
        p2_s[...] = 1.0 - p1

    gate = (jnp.where(i1_s[...] == e, p1_s[...], 0.0)
            + jnp.where(i2_s[...] == e, p2_s[...], 0.0))

    lora = lax.dot_general(
        al_ref[0].astype(jnp.bfloat16), bl_ref[0].astype(jnp.bfloat16),
        (((1,), (0,)), ((), ())), preferred_element_type=jnp.float32)
    weff = (we_ref[0] + (_ALPHA / _RANK) * lora).astype(jnp.bfloat16)
    yo = lax.dot_general(
        hfb_s[...], weff, (((1,), (0,)), ((), ())),
        preferred_element_type=jnp.float32) + be_ref[0]
    contrib = gate * jax.nn.gelu(yo)

    @pl.when(e == 0)
    def _init():
        out_ref[...] = hf_ref[...] + contrib

    @pl.when(e > 0)
    def _acc():
        out_ref[...] += contrib


def _moe(hf, Wr, We, be3, Al, Bl):
    return pl.pallas_call(
        _moe_body,
        grid=(_N // _NT, _E),
        in_specs=[
            pl.BlockSpec((_NT, _D), lambda n, e: (n, 0)),
            pl.BlockSpec((_D, _E), lambda n, e: (0, 0)),
            pl.BlockSpec((1, _D, _D), lambda n, e: (e, 0, 0)),
            pl.BlockSpec((1, 1, _D), lambda n, e: (e, 0, 0)),
            pl.BlockSpec((1, _D, _RANK), lambda n, e: (e, 0, 0)),
            pl.BlockSpec((1, _RANK, _D), lambda n, e: (e, 0, 0)),
        ],
        out_specs=pl.BlockSpec((_NT, _D), lambda n, e: (n, 0)),
        out_shape=jax.ShapeDtypeStruct((_N, _D), jnp.float32),
        scratch_shapes=[
            pltpu.VMEM((_NT, 1), jnp.int32),
            pltpu.VMEM((_NT, 1), jnp.int32),
            pltpu.VMEM((_NT, 1), jnp.float32),
            pltpu.VMEM((_NT, 1), jnp.float32),
            pltpu.VMEM((_NT, _D), jnp.bfloat16),
        ],
        compiler_params=pltpu.CompilerParams(
            dimension_semantics=("arbitrary", "arbitrary")),
    )(hf, Wr, We, be3, Al, Bl)


def _proj_body(res_ref, wfc_ref, bfc_ref, out_ref):
    res = res_ref[...].astype(jnp.bfloat16)
    w = wfc_ref[...].astype(jnp.bfloat16)
    acc = lax.dot_general(res, w, (((1,), (0,)), ((), ())),
                          preferred_element_type=jnp.float32)
    out_ref[...] = acc + bfc_ref[...]


def _proj(res, Wfc, bfc2):
    return pl.pallas_call(
        _proj_body,
        grid=(_VOCAB // _VB,),
        in_specs=[
            pl.BlockSpec((_N, _D), lambda v: (0, 0)),
            pl.BlockSpec((_D, _VB), lambda v: (0, v)),
            pl.BlockSpec((1, _VB), lambda v: (0, v)),
        ],
        out_specs=pl.BlockSpec((_N, _VB), lambda v: (0, v)),
        out_shape=jax.ShapeDtypeStruct((_N, _VOCAB), jnp.float32),
        compiler_params=pltpu.CompilerParams(
            dimension_semantics=("arbitrary",)),
    )(res, Wfc, bfc2)


def kernel(x, emb, Wr, We, be, Al, Bl, Wfc, bfc):
    B, S = x.shape
    hf = _emb_gather(x.reshape(-1), emb)
    res = _moe(hf, Wr, We, be.reshape(_E, 1, _D), Al, Bl)
    out = _proj(res, Wfc, bfc.reshape(1, _VOCAB))
    return out.reshape(B, S, _VOCAB)


# speedup vs baseline: 1.3381x; 1.0188x over previous
import functools

import jax
import jax.numpy as jnp
from jax import lax
from jax.experimental import pallas as pl
from jax.experimental.pallas import tpu as pltpu
from jax.experimental.pallas import tpu_sc as plsc

_VOCAB = 32000
_D = 1024
_E = 8
_RANK = 4
_ALPHA = 1.0
_N = 2048

_NC = 2
_NS = 16
_NW = _NC * _NS
_BPW = _N // _NW

_VB = 640
_NH = 2
_NEG = -1e30


def _emb_gather(x_flat, emb):
    mesh = plsc.VectorSubcoreMesh(core_axis_name="c", subcore_axis_name="s")

    @functools.partial(
        pl.kernel,
        mesh=mesh,
        out_type=jax.ShapeDtypeStruct((_N, _D), jnp.float32),
        scratch_types=[
            pltpu.VMEM((_BPW,), jnp.int32),
            pltpu.VMEM((_BPW, _D), jnp.float32),
            pltpu.SemaphoreType.DMA,
        ],
    )
    def gather_k(idx_hbm, table_hbm, out_hbm, idx_v, rows_v, sem):
        wid = lax.axis_index("s") * _NC + lax.axis_index("c")
        base = wid * _BPW
        pltpu.sync_copy(idx_hbm.at[pl.ds(base, _BPW)], idx_v)
        pltpu.async_copy(table_hbm.at[idx_v], rows_v, sem).wait()
        pltpu.sync_copy(rows_v, out_hbm.at[pl.ds(base, _BPW)])

    return gather_k(x_flat, emb)


def _fused_body(hf_ref, wr_ref, we_ref, be_ref, al_ref, bl_ref, wfc_ref,
                bfc_ref, out_ref, i1_s, i2_s, p1_s, p2_s, acc_s, resb_s):
    i = pl.program_id(0)

    @pl.when(i == 0)
    def _router():
        hfb = hf_ref[...].astype(jnp.bfloat16)
        logits = lax.dot_general(
            hfb, wr_ref[...].astype(jnp.bfloat16), (((1,), (0,)), ((), ())),
            preferred_element_type=jnp.float32)
        idx = lax.broadcasted_iota(jnp.int32, (_N, _E), 1)
        m1 = jnp.max(logits, axis=1, keepdims=True)
        i1 = jnp.min(jnp.where(logits == m1, idx, _E), axis=1, keepdims=True)
        l2 = jnp.where(idx == i1, _NEG, logits)
        m2 = jnp.max(l2, axis=1, keepdims=True)
        i2 = jnp.min(jnp.where(l2 == m2, idx, _E), axis=1, keepdims=True)
        p1 = 1.0 / (1.0 + jnp.exp(m2 - m1))
        i1_s[...] = i1
        i2_s[...] = i2
        p1_s[...] = p1
        p2_s[...] = 1.0 - p1

    @pl.when(i < _E)
    def _expert():
        e = i
        lora = lax.dot_general(
            al_ref[0].astype(jnp.bfloat16), bl_ref[0].astype(jnp.bfloat16),
            (((1,), (0,)), ((), ())), preferred_element_type=jnp.float32)
        weff = (we_ref[0] + (_ALPHA / _RANK) * lora).astype(jnp.bfloat16)
        gate = (jnp.where(i1_s[...] == e, p1_s[...], 0.0)
                + jnp.where(i2_s[...] == e, p2_s[...], 0.0))
        for h in range(_NH):
            lo, hi = h * (_N // _NH), (h + 1) * (_N // _NH)
            sl = slice(lo, hi)
            yo = lax.dot_general(
                hf_ref[sl, :].astype(jnp.bfloat16), weff,
                (((1,), (0,)), ((), ())),
                preferred_element_type=jnp.float32) + be_ref[0]
            contrib = gate[sl, :] * jax.nn.gelu(yo)

            @pl.when(e == 0)
            def _init():
                acc_s[sl, :] = contrib

            @pl.when(e > 0)
            def _acc():
                acc_s[sl, :] += contrib

        @pl.when(e == _E - 1)
        def _finalize():
            resb_s[...] = (hf_ref[...] + acc_s[...]).astype(jnp.bfloat16)

    @pl.when(i >= _E)
    def _proj():
        out_ref[...] = lax.dot_general(
            resb_s[...], wfc_ref[...].astype(jnp.bfloat16),
            (((1,), (0,)), ((), ())),
            preferred_element_type=jnp.float32) + bfc_ref[...]


def _fused(hf, Wr, We, be3, Al, Bl, Wfc, bfc2):
    nv = _VOCAB // _VB

    def expert_ix(i):
        return jnp.minimum(i, _E - 1)

    def vocab_ix(i):
        return jnp.maximum(i - _E, 0)

    return pl.pallas_call(
        _fused_body,
        grid=(_E + nv,),
        in_specs=[
            pl.BlockSpec((_N, _D), lambda i: (0, 0)),
            pl.BlockSpec((_D, _E), lambda i: (0, 0)),
            pl.BlockSpec((1, _D, _D), lambda i: (expert_ix(i), 0, 0)),
            pl.BlockSpec((1, 1, _D), lambda i: (expert_ix(i), 0, 0)),
            pl.BlockSpec((1, _D, _RANK), lambda i: (expert_ix(i), 0, 0)),
            pl.BlockSpec((1, _RANK, _D), lambda i: (expert_ix(i), 0, 0)),
            pl.BlockSpec((_D, _VB), lambda i: (0, vocab_ix(i))),
            pl.BlockSpec((1, _VB), lambda i: (0, vocab_ix(i))),
        ],
        out_specs=pl.BlockSpec((_N, _VB), lambda i: (0, vocab_ix(i))),
        out_shape=jax.ShapeDtypeStruct((_N, _VOCAB), jnp.float32),
        scratch_shapes=[
            pltpu.VMEM((_N, 1), jnp.int32),
            pltpu.VMEM((_N, 1), jnp.int32),
            pltpu.VMEM((_N, 1), jnp.float32),
            pltpu.VMEM((_N, 1), jnp.float32),
            pltpu.VMEM((_N, _D), jnp.float32),
            pltpu.VMEM((_N, _D), jnp.bfloat16),
        ],
        compiler_params=pltpu.CompilerParams(
            dimension_semantics=("arbitrary",)),
    )(hf, Wr, We, be3, Al, Bl, Wfc, bfc2)


def kernel(x, emb, Wr, We, be, Al, Bl, Wfc, bfc):
    B, S = x.shape
    hf = _emb_gather(x.reshape(-1), emb)
    out = _fused(hf, Wr, We, be.reshape(_E, 1, _D), Al, Bl,
                 Wfc, bfc.reshape(1, _VOCAB))
    return out.reshape(B, S, _VOCAB)


# speedup vs baseline: 1.3619x; 1.0178x over previous
import functools

import jax
import jax.numpy as jnp
from jax import lax
from jax.experimental import pallas as pl
from jax.experimental.pallas import tpu as pltpu
from jax.experimental.pallas import tpu_sc as plsc

_VOCAB = 32000
_D = 1024
_E = 8
_RANK = 4
_ALPHA = 1.0
_N = 2048

_NC = 2
_NS = 16
_NW = _NC * _NS
_BPW = _N // _NW

_VB = 640
_NH = 2
_NEG = -1e30


def _emb_gather(x_flat, emb):
    mesh = plsc.VectorSubcoreMesh(core_axis_name="c", subcore_axis_name="s")

    @functools.partial(
        pl.kernel,
        mesh=mesh,
        out_type=jax.ShapeDtypeStruct((_N, _D), jnp.float32),
        scratch_types=[
            pltpu.VMEM((_BPW,), jnp.int32),
            pltpu.VMEM((_BPW, _D), jnp.float32),
            pltpu.SemaphoreType.DMA,
        ],
    )
    def gather_k(idx_hbm, table_hbm, out_hbm, idx_v, rows_v, sem):
        wid = lax.axis_index("s") * _NC + lax.axis_index("c")
        base = wid * _BPW
        pltpu.sync_copy(idx_hbm.at[pl.ds(base, _BPW)], idx_v)
        pltpu.async_copy(table_hbm.at[idx_v], rows_v, sem).wait()
        pltpu.sync_copy(rows_v, out_hbm.at[pl.ds(base, _BPW)])

    return gather_k(x_flat, emb)


def _fused_body(hf_ref, wr_ref, we_ref, be_ref, al_ref, bl_ref, wfc_ref,
                bfc_ref, out_ref, rt_s, acc_s):
    i = pl.program_id(0)

    @pl.when(i == 0)
    def _router():
        hfb = hf_ref[...].astype(jnp.bfloat16)
        logits = lax.dot_general(
            hfb, wr_ref[...].astype(jnp.bfloat16), (((1,), (1,)), ((), ())),
            preferred_element_type=jnp.float32)
        idx = lax.broadcasted_iota(jnp.int32, (_N, _E), 1)
        m1 = jnp.max(logits, axis=1, keepdims=True)
        i1 = jnp.min(jnp.where(logits == m1, idx, _E), axis=1, keepdims=True)
        l2 = jnp.where(idx == i1, _NEG, logits)
        m2 = jnp.max(l2, axis=1, keepdims=True)
        i2 = jnp.min(jnp.where(l2 == m2, idx, _E), axis=1, keepdims=True)
        p1 = 1.0 / (1.0 + jnp.exp(m2 - m1))
        rt_s[...] = jnp.concatenate(
            [i1.astype(jnp.float32), i2.astype(jnp.float32), p1, 1.0 - p1],
            axis=1)

    @pl.when(i < _E)
    def _expert():
        e = i
        ef = e.astype(jnp.float32)
        lora = lax.dot_general(
            al_ref[0].astype(jnp.bfloat16), bl_ref[0].astype(jnp.bfloat16),
            (((0,), (0,)), ((), ())), preferred_element_type=jnp.float32)
        weff = (we_ref[0] + (_ALPHA / _RANK) * lora).astype(jnp.bfloat16)
        rt = rt_s[...]
        gate = (jnp.where(rt[:, 0:1] == ef, rt[:, 2:3], 0.0)
                + jnp.where(rt[:, 1:2] == ef, rt[:, 3:4], 0.0))
        for h in range(_NH):
            sl = slice(h * (_N // _NH), (h + 1) * (_N // _NH))
            yo = lax.dot_general(
                hf_ref[sl, :].astype(jnp.bfloat16), weff,
                (((1,), (0,)), ((), ())),
                preferred_element_type=jnp.float32) + be_ref[0]
            contrib = gate[sl, :] * jax.nn.gelu(yo)

            @pl.when(e == 0)
            def _init():
                acc_s[sl, :] = contrib.astype(jnp.bfloat16)

            @pl.when((e > 0) & (e < _E - 1))
            def _acc():
                acc_s[sl, :] = (acc_s[sl, :].astype(jnp.float32)
                                + contrib).astype(jnp.bfloat16)

            @pl.when(e == _E - 1)
            def _residual():
                acc_s[sl, :] = (hf_ref[sl, :] + acc_s[sl, :].astype(jnp.float32)
                                + contrib).astype(jnp.bfloat16)

    @pl.when(i >= _E)
    def _proj():
        out_ref[...] = lax.dot_general(
            acc_s[...], wfc_ref[...].astype(jnp.bfloat16),
            (((1,), (0,)), ((), ())),
            preferred_element_type=jnp.float32) + bfc_ref[...]


def _fused(hf, Wr, We, be3, Al, Bl, Wfc, bfc2):
    nv = _VOCAB // _VB

    def expert_ix(i):
        return jnp.minimum(i, _E - 1)

    def vocab_ix(i):
        return jnp.maximum(i - _E, 0)

    return pl.pallas_call(
        _fused_body,
        grid=(_E + nv,),
        in_specs=[
            pl.BlockSpec((_N, _D), lambda i: (0, 0)),
            pl.BlockSpec((_E, _D), lambda i: (0, 0)),
            pl.BlockSpec((1, _D, _D), lambda i: (expert_ix(i), 0, 0)),
            pl.BlockSpec((1, 1, _D), lambda i: (expert_ix(i), 0, 0)),
            pl.BlockSpec((1, _RANK, _D), lambda i: (expert_ix(i), 0, 0)),
            pl.BlockSpec((1, _RANK, _D), lambda i: (expert_ix(i), 0, 0)),
            pl.BlockSpec((_D, _VB), lambda i: (0, vocab_ix(i))),
            pl.BlockSpec((1, _VB), lambda i: (0, vocab_ix(i))),
        ],
        out_specs=pl.BlockSpec((_N, _VB), lambda i: (0, vocab_ix(i))),
        out_shape=jax.ShapeDtypeStruct((_N, _VOCAB), jnp.float32),
        scratch_shapes=[
            pltpu.VMEM((_N, 4), jnp.float32),
            pltpu.VMEM((_N, _D), jnp.bfloat16),
        ],
        compiler_params=pltpu.CompilerParams(
            dimension_semantics=("arbitrary",)),
    )(hf, Wr, We, be3, Al, Bl, Wfc, bfc2)


def kernel(x, emb, Wr, We, be, Al, Bl, Wfc, bfc):
    B, S = x.shape
    hf = _emb_gather(x.reshape(-1), emb)
    out = _fused(hf, Wr.T, We, be.reshape(_E, 1, _D),
                 jnp.swapaxes(Al, 1, 2), Bl, Wfc, bfc.reshape(1, _VOCAB))
    return out.reshape(B, S, _VOCAB)
